# Initial kernel scaffold; baseline (speedup 1.0000x reference)
import jax
import jax.numpy as jnp
from jax.experimental import pallas as pl


def kernel(inputs):
    raise NotImplementedError("write your pallas kernel here")


# speedup vs baseline: 37.2294x; 37.2294x over previous
import functools

import jax
import jax.numpy as jnp
from jax import lax
from jax.experimental import pallas as pl
from jax.experimental.pallas import tpu as pltpu, tpu_sc as plsc

N = 4096
B = 4
TOTAL = N * (N - 1) // 2
NW = 32
SPAN = TOTAL // NW
CHUNK = 8736
NCHUNK = SPAN // CHUNK
VECS = CHUNK // 16


def _body(in_hbm, out_hbm, in_v, buf_v):
    wid = lax.axis_index("c") * 16 + lax.axis_index("s")
    pltpu.sync_copy(in_hbm, in_v)

    lane = lax.iota(jnp.int32, 16)

    def chunk_body(m, _):
        p0 = wid * SPAN + m * CHUNK

        def vec_body(k, _):
            p = p0 + k * 16 + lane
            lo = jnp.zeros((16,), jnp.int32)
            hi = jnp.full((16,), N - 1, jnp.int32)
            for _step in range(12):
                mid = (lo + hi) >> 1
                off_mid = (mid * ((2 * N - 1) - mid)) >> 1
                le = off_mid <= p
                lo = jnp.where(le, mid, lo)
                hi = jnp.where(le, hi, mid)
            i_idx = lo
            off_i = (i_idx * ((2 * N - 1) - i_idx)) >> 1
            j_idx = p - off_i + i_idx + 1
            for b in range(B):
                vj = plsc.load_gather(in_v, [b * N + j_idx])
                vi = plsc.load_gather(in_v, [b * N + i_idx])
                buf_v[pl.ds(b * CHUNK + k * 16, 16)] = vj - vi
            return 0

        lax.fori_loop(0, VECS, vec_body, 0)
        for b in range(B):
            pltpu.sync_copy(
                buf_v.at[pl.ds(b * CHUNK, CHUNK)],
                out_hbm.at[pl.ds(b * TOTAL + p0, CHUNK)],
            )
        return 0

    lax.fori_loop(0, NCHUNK, chunk_body, 0)


@jax.jit
def kernel(inputs):
    mesh = plsc.VectorSubcoreMesh(core_axis_name="c", subcore_axis_name="s")
    f = pl.kernel(
        _body,
        out_type=jax.ShapeDtypeStruct((B * TOTAL,), jnp.float32),
        mesh=mesh,
        compiler_params=pltpu.CompilerParams(needs_layout_passes=False),
        scratch_types=[
            pltpu.VMEM((B * N,), jnp.float32),
            pltpu.VMEM((B * CHUNK,), jnp.float32),
        ],
    )
    return f(inputs.reshape(B * N)).reshape(B, TOTAL)
